# baseline (speedup 1.0000x reference)
import functools

import jax
import jax.numpy as jnp
from jax import lax
from jax.experimental import pallas as pl
from jax.experimental.pallas import tpu as pltpu
from jax.experimental.pallas import tpu_sc as plsc

N = 8192
K = 8192
D = 256
L = 128
BN = 1024
CG = 1024
NI = N // BN
NG = K // CG
NJ = CG // L
LOSS_SCALE = 1.25 / (N * D)


def _c2_body(c_ref, c2_ref):
    c = c_ref[...]
    c2_ref[...] = jnp.sum(c * c, axis=1).reshape(K // L, L)


_c2_call = pl.pallas_call(
    _c2_body,
    out_shape=jax.ShapeDtypeStruct((K // L, L), jnp.float32),
)


def _argmin_body(x_ref, c_ref, c2_ref, minidx_ref, loss_ref):
    i = pl.program_id(0)
    x = x_ref[...]
    x2 = jnp.sum(x * x, axis=1, keepdims=True)
    x2b = jnp.broadcast_to(x2, (BN, L))
    xx = x + x
    lane = lax.broadcasted_iota(jnp.int32, (1, L), 1)

    cur = jnp.full((BN, L), jnp.inf, jnp.float32)
    curc = jnp.zeros((BN, L), jnp.int32)
    for g in range(NG):
        cg = c_ref[g * CG:(g + 1) * CG, :]
        xc2 = lax.dot_general(xx, cg, (((1,), (1,)), ((), ())),
                              preferred_element_type=jnp.float32)
        for j in range(NJ):
            dj = (x2b - xc2[:, j * L:(j + 1) * L]) + c2_ref[g * NJ + j][None, :]
            lt = dj < cur
            curc = jnp.where(lt, lane + (g * CG + j * L), curc)
            cur = jnp.minimum(dj, cur)

    bmin = jnp.min(cur, axis=1, keepdims=True)
    bidx = jnp.min(jnp.where(cur == bmin, curc, K), axis=1, keepdims=True)
    minidx_ref[...] = bidx.reshape(BN // L, L)
    s = jnp.sum(bmin).reshape(1, 1)

    @pl.when(i == 0)
    def _():
        loss_ref[...] = s

    @pl.when(i > 0)
    def _():
        loss_ref[...] = loss_ref[...] + s

    @pl.when(i == NI - 1)
    def _():
        loss_ref[...] = loss_ref[...] * LOSS_SCALE


_argmin_call = pl.pallas_call(
    _argmin_body,
    grid=(NI,),
    in_specs=[
        pl.BlockSpec((BN, D), lambda i: (i, 0)),
        pl.BlockSpec((K, D), lambda i: (0, 0)),
        pl.BlockSpec((K // L, L), lambda i: (0, 0)),
    ],
    out_specs=[
        pl.BlockSpec((BN // L, L), lambda i: (i, 0)),
        pl.BlockSpec((1, 1), lambda i: (0, 0)),
    ],
    out_shape=[
        jax.ShapeDtypeStruct((N // L, L), jnp.int32),
        jax.ShapeDtypeStruct((1, 1), jnp.float32),
    ],
)


@functools.cache
def _make_sc_gather():
    info = plsc.get_sparse_core_info()
    nw = info.num_cores * info.num_subcores
    b_per_w = N // nw
    mesh = plsc.VectorSubcoreMesh(core_axis_name="c", subcore_axis_name="s")

    @functools.partial(
        pl.kernel, mesh=mesh,
        out_type=jax.ShapeDtypeStruct((N, D), jnp.float32),
        scratch_types=[
            pltpu.VMEM((b_per_w,), jnp.int32),
            pltpu.VMEM((b_per_w, D), jnp.float32),
            pltpu.SemaphoreType.DMA,
        ],
    )
    def gather(table_hbm, idx_hbm, out_hbm, idx_v, rows_v, sem):
        wid = lax.axis_index("s") * info.num_cores + lax.axis_index("c")
        base = wid * b_per_w
        pltpu.sync_copy(idx_hbm.at[pl.ds(base, b_per_w)], idx_v)
        pltpu.async_copy(table_hbm.at[idx_v], rows_v, sem).wait()
        pltpu.sync_copy(rows_v, out_hbm.at[pl.ds(base, b_per_w)])

    return gather


def kernel(inputs, codebook):
    flat = inputs.reshape(N, D)
    c2 = _c2_call(codebook)
    minidx, loss = _argmin_call(flat, codebook, c2)
    quantized = _make_sc_gather()(codebook, minidx.reshape(N))
    return quantized.reshape(inputs.shape), loss.reshape(())


# speedup vs baseline: 1.0083x; 1.0083x over previous
import functools

import jax
import jax.numpy as jnp
from jax import lax
from jax.experimental import pallas as pl
from jax.experimental.pallas import tpu as pltpu
from jax.experimental.pallas import tpu_sc as plsc

N = 8192
K = 8192
D = 256
L = 128
BN = 2048
CG = 1024
NI = N // BN
NG = K // CG
NJ = CG // L
LOSS_SCALE = 1.25 / (N * D)


def _c2_body(c_ref, c2_ref):
    c = c_ref[...]
    c2_ref[...] = jnp.sum(c * c, axis=1).reshape(K // L, L)


_c2_call = pl.pallas_call(
    _c2_body,
    out_shape=jax.ShapeDtypeStruct((K // L, L), jnp.float32),
)


def _argmin_body(x_ref, c_ref, c2_ref, minidx_ref, loss_ref):
    i = pl.program_id(0)
    x = x_ref[...]
    x2 = jnp.sum(x * x, axis=1, keepdims=True)
    x2b = jnp.broadcast_to(x2, (BN, L))
    xx = x + x
    lane = lax.broadcasted_iota(jnp.int32, (1, L), 1)

    cur = jnp.full((BN, L), jnp.inf, jnp.float32)
    curc = jnp.zeros((BN, L), jnp.int32)
    for g in range(NG):
        cg = c_ref[g * CG:(g + 1) * CG, :]
        xc2 = lax.dot_general(xx, cg, (((1,), (1,)), ((), ())),
                              preferred_element_type=jnp.float32)
        for j in range(NJ):
            dj = (x2b - xc2[:, j * L:(j + 1) * L]) + c2_ref[g * NJ + j][None, :]
            lt = dj < cur
            curc = jnp.where(lt, lane + (g * CG + j * L), curc)
            cur = jnp.minimum(dj, cur)

    bmin = jnp.min(cur, axis=1, keepdims=True)
    bidx = jnp.min(jnp.where(cur == bmin, curc, K), axis=1, keepdims=True)
    minidx_ref[...] = bidx.reshape(BN // L, L)
    s = jnp.sum(bmin).reshape(1, 1)

    @pl.when(i == 0)
    def _():
        loss_ref[...] = s

    @pl.when(i > 0)
    def _():
        loss_ref[...] = loss_ref[...] + s

    @pl.when(i == NI - 1)
    def _():
        loss_ref[...] = loss_ref[...] * LOSS_SCALE


_argmin_call = pl.pallas_call(
    _argmin_body,
    grid=(NI,),
    in_specs=[
        pl.BlockSpec((BN, D), lambda i: (i, 0)),
        pl.BlockSpec((K, D), lambda i: (0, 0)),
        pl.BlockSpec((K // L, L), lambda i: (0, 0)),
    ],
    out_specs=[
        pl.BlockSpec((BN // L, L), lambda i: (i, 0)),
        pl.BlockSpec((1, 1), lambda i: (0, 0)),
    ],
    out_shape=[
        jax.ShapeDtypeStruct((N // L, L), jnp.int32),
        jax.ShapeDtypeStruct((1, 1), jnp.float32),
    ],
)


@functools.cache
def _make_sc_gather():
    info = plsc.get_sparse_core_info()
    nw = info.num_cores * info.num_subcores
    b_per_w = N // nw
    mesh = plsc.VectorSubcoreMesh(core_axis_name="c", subcore_axis_name="s")

    @functools.partial(
        pl.kernel, mesh=mesh,
        out_type=jax.ShapeDtypeStruct((N, D), jnp.float32),
        scratch_types=[
            pltpu.VMEM((b_per_w,), jnp.int32),
            pltpu.VMEM((b_per_w, D), jnp.float32),
            pltpu.SemaphoreType.DMA,
        ],
    )
    def gather(table_hbm, idx_hbm, out_hbm, idx_v, rows_v, sem):
        wid = lax.axis_index("s") * info.num_cores + lax.axis_index("c")
        base = wid * b_per_w
        pltpu.sync_copy(idx_hbm.at[pl.ds(base, b_per_w)], idx_v)
        pltpu.async_copy(table_hbm.at[idx_v], rows_v, sem).wait()
        pltpu.sync_copy(rows_v, out_hbm.at[pl.ds(base, b_per_w)])

    return gather


def kernel(inputs, codebook):
    flat = inputs.reshape(N, D)
    c2 = _c2_call(codebook)
    minidx, loss = _argmin_call(flat, codebook, c2)
    quantized = _make_sc_gather()(codebook, minidx.reshape(N))
    return quantized.reshape(inputs.shape), loss.reshape(())
